# baseline (speedup 1.0000x reference)
import functools

import jax
import jax.numpy as jnp
from jax import lax
from jax.experimental import pallas as pl
from jax.experimental.pallas import tpu as pltpu
from jax.experimental.pallas import tpu_sc as plsc

N_NODES = 10000
N_EDGES = 320000
D_FEAT = 128
D_EDGE = 16
D_OUT = 16
N_PK = N_EDGES // 8


def _nodeproj_body(x_ref, w_ref, xs_ref, xd_ref):
    xs_ref[...] = jnp.dot(x_ref[...], w_ref[D_EDGE:D_EDGE + D_FEAT, :],
                          preferred_element_type=jnp.float32)
    xd_ref[...] = jnp.dot(x_ref[...], w_ref[D_EDGE + D_FEAT:, :],
                          preferred_element_type=jnp.float32)


def _nodeproj(x, W):
    return pl.pallas_call(
        _nodeproj_body,
        out_shape=[
            jax.ShapeDtypeStruct((N_NODES, D_OUT), jnp.float32),
            jax.ShapeDtypeStruct((N_NODES, D_OUT), jnp.float32),
        ],
    )(x, W)


_EBLK = 8000
_NBLK = N_EDGES // _EBLK
_PBLK = _EBLK // 8


def _eproj_body(ea_ref, w_ref, b_ref, out_ref):
    t = (jnp.dot(ea_ref[...], w_ref[:D_EDGE, :],
                 preferred_element_type=jnp.float32) + b_ref[...])
    out_ref[...] = jnp.concatenate(
        [t[j * _PBLK:(j + 1) * _PBLK, :] for j in range(8)], axis=1)


def _eproj(edge_attr, W, b2d):
    return pl.pallas_call(
        _eproj_body,
        grid=(_NBLK,),
        in_specs=[
            pl.BlockSpec((_EBLK, D_EDGE), lambda i: (i, 0)),
            pl.BlockSpec((D_EDGE + 2 * D_FEAT, D_OUT), lambda i: (0, 0)),
            pl.BlockSpec((1, D_OUT), lambda i: (0, 0)),
        ],
        out_specs=pl.BlockSpec((_PBLK, 128), lambda i: (i, 0)),
        out_shape=jax.ShapeDtypeStruct((N_PK, 128), jnp.float32),
    )(edge_attr, W, b2d)


def _unpack_body(in_ref, out_ref):
    p = in_ref[...]
    out_ref[...] = jnp.concatenate(
        [p[:, j * D_OUT:(j + 1) * D_OUT] for j in range(8)], axis=0)


def _unpack(packed):
    return pl.pallas_call(
        _unpack_body,
        grid=(_NBLK,),
        in_specs=[pl.BlockSpec((_PBLK, 128), lambda i: (i, 0))],
        out_specs=pl.BlockSpec((_EBLK, D_OUT), lambda i: (i, 0)),
        out_shape=jax.ShapeDtypeStruct((N_EDGES, D_OUT), jnp.float32),
    )(packed)


_NW = 32
_EPW = N_EDGES // _NW
_MACRO = 2000
_PMACRO = _MACRO // 8
_NMACRO = _EPW // _MACRO

_mesh = plsc.VectorSubcoreMesh(core_axis_name="c", subcore_axis_name="s")


@functools.partial(
    pl.kernel,
    mesh=_mesh,
    compiler_params=pltpu.CompilerParams(use_tc_tiling_on_sc=False),
    out_type=jax.ShapeDtypeStruct((N_PK, 128), jnp.float32),
    scratch_types=[
        pltpu.VMEM((_EPW,), jnp.int32),
        pltpu.VMEM((_EPW,), jnp.int32),
        pltpu.VMEM((_MACRO, D_OUT), jnp.float32),
        pltpu.VMEM((_MACRO, D_OUT), jnp.float32),
        pltpu.VMEM((_PMACRO, 128), jnp.float32),
        pltpu.SemaphoreType.DMA,
        pltpu.SemaphoreType.DMA,
        pltpu.SemaphoreType.DMA,
    ],
)
def _sc_gather_add(src_hbm, dst_hbm, xs_hbm, xd_hbm, ea_hbm, out_hbm,
                   idx_s, idx_d, rows_s, rows_d, acc, sem_s, sem_d, sem_e):
    wid = lax.axis_index("s") * 2 + lax.axis_index("c")
    base = wid * _EPW
    pltpu.sync_copy(src_hbm.at[pl.ds(base, _EPW)], idx_s)
    pltpu.sync_copy(dst_hbm.at[pl.ds(base, _EPW)], idx_d)
    for m in range(_NMACRO):
        off = m * _MACRO
        poff = (base + off) // 8
        cp_e = pltpu.async_copy(ea_hbm.at[pl.ds(poff, _PMACRO), :], acc, sem_e)
        cp_s = pltpu.async_copy(xs_hbm.at[idx_s.at[pl.ds(off, _MACRO)]], rows_s, sem_s)
        cp_d = pltpu.async_copy(xd_hbm.at[idx_d.at[pl.ds(off, _MACRO)]], rows_d, sem_d)
        cp_e.wait()
        cp_s.wait()
        cp_d.wait()

        def body(r2, _):
            for k in range(8):
                sl = pl.ds(k * D_OUT, D_OUT)
                acc[r2, sl] = (acc[r2, sl]
                               + rows_s[r2 * 8 + k, :] + rows_d[r2 * 8 + k, :])
            return 0

        lax.fori_loop(0, _PMACRO, body, 0)
        pltpu.sync_copy(acc, out_hbm.at[pl.ds(poff, _PMACRO), :])


def _to_packed_order(v):
    return v.reshape(_NBLK, 8, _PBLK).transpose(0, 2, 1).reshape(N_EDGES)


def kernel(x, edge_index, edge_attr, pos, W, b):
    src = _to_packed_order(edge_index[0])
    dst = _to_packed_order(edge_index[1])
    xs, xd = _nodeproj(x, W)
    eap = _eproj(edge_attr, W, b.reshape(1, D_OUT))
    out_pk = _sc_gather_add(src, dst, xs, xd, eap)
    return (x, _unpack(out_pk), edge_index, pos)
